# baseline (speedup 1.0000x reference)
import functools

import jax
import jax.numpy as jnp
from jax import lax
from jax.experimental import pallas as pl
from jax.experimental.pallas import tpu as pltpu
from jax.experimental.pallas import tpu_sc as plsc

_BATCH = 16384
_EMB_K = 32
_ROWS = 1000000
_BU = 32768
_NBT = (_ROWS + _BU - 1) // _BU
_SROWS = _NBT * _BU // 128
_SFLAT = _SROWS * 128

_C = 1024
_KG = 4
_KS = 8
_B_END = 30 * _BU
_F_SC = 12 * 32 * _C
_A_BEG = _B_END - _F_SC


def _tc_fold(wt_ref, ht_ref, wu_ref, wv_ref, sw_ref, sh_ref):
    wu = wu_ref[...]
    wv = wv_ref[...]
    wb = wt_ref[...]
    hb = ht_ref[...]
    for j in range(_BU // 128):
        sl = slice(j * 128, (j + 1) * 128)
        sw_ref[j, :] = jnp.sum(wb[:, sl] * wu, axis=0)
        sh_ref[j, :] = jnp.sum(hb[:, sl] * wv, axis=0)


def _fold_tables_tc(wt, ht, wu, wv):
    n1 = _A_BEG // _BU
    n2 = _NBT - _B_END // _BU
    off = _B_END // _BU - n1

    def imap_in(i):
        j = jnp.where(i < n1, i, i + off)
        return (0, j)

    def imap_out(i):
        j = jnp.where(i < n1, i, i + off)
        return (j, 0)

    sw, sh = pl.pallas_call(
        _tc_fold,
        grid=(n1 + n2,),
        in_specs=[
            pl.BlockSpec((_EMB_K, _BU), imap_in),
            pl.BlockSpec((_EMB_K, _BU), imap_in),
            pl.BlockSpec((_EMB_K, 128), lambda i: (0, 0)),
            pl.BlockSpec((_EMB_K, 128), lambda i: (0, 0)),
        ],
        out_specs=[
            pl.BlockSpec((_BU // 128, 128), imap_out),
            pl.BlockSpec((_BU // 128, 128), imap_out),
        ],
        out_shape=[
            jax.ShapeDtypeStruct((_SROWS, 128), jnp.float32),
            jax.ShapeDtypeStruct((_SROWS, 128), jnp.float32),
        ],
        compiler_params=pltpu.CompilerParams(
            dimension_semantics=("arbitrary",)),
    )(wt, ht, wu, wv)
    return sw.reshape(_SFLAT), sh.reshape(_SFLAT)


def _build_sc_fold():
    info = plsc.get_sparse_core_info()
    nc, ns, lanes = info.num_cores, info.num_subcores, info.num_lanes
    nw = nc * ns
    per_w = _F_SC // nw
    n_chunks = per_w // _C

    mesh = plsc.VectorSubcoreMesh(core_axis_name="c", subcore_axis_name="s")

    @functools.partial(
        pl.kernel,
        out_type=[jax.ShapeDtypeStruct((_SFLAT,), jnp.float32),
                  jax.ShapeDtypeStruct((_SFLAT,), jnp.float32)],
        mesh=mesh,
        compiler_params=pltpu.CompilerParams(
            needs_layout_passes=False, use_tc_tiling_on_sc=True),
        scratch_types=[
            pltpu.VMEM((_KG * _KS, _C), jnp.float32),
            pltpu.VMEM((_KG * _KS, _C), jnp.float32),
            pltpu.VMEM((2 * _EMB_K,), jnp.float32),
            pltpu.VMEM((_C,), jnp.float32),
            pltpu.VMEM((_C,), jnp.float32),
            pltpu.SemaphoreType.DMA,
        ],
    )
    def fold(wt4_hbm, ht4_hbm, w_hbm, sw_hbm, sh_hbm,
             bufw, bufh, wvm, sbw, sbh, sem):
        wid = lax.axis_index("s") * nc + lax.axis_index("c")
        base = _A_BEG + wid * per_w
        pltpu.sync_copy(w_hbm, wvm)
        wregs = [wvm[pl.ds(j * lanes, lanes)]
                 for j in range(2 * _EMB_K // lanes)]
        wk = [wregs[k // lanes][k % lanes] for k in range(2 * _EMB_K)]

        def chunk_body(ci, carry):
            u0 = base + ci * _C
            cps = []
            for g in range(_KG):
                cps.append(pltpu.async_copy(
                    wt4_hbm.at[g, :, pl.ds(u0, _C)],
                    bufw.at[pl.ds(g * _KS, _KS)], sem))
                cps.append(pltpu.async_copy(
                    ht4_hbm.at[g, :, pl.ds(u0, _C)],
                    bufh.at[pl.ds(g * _KS, _KS)], sem))
            for cp in cps:
                cp.wait()
            def col_body(j, c2):
                sl = pl.ds(j * lanes, lanes)
                accw = bufw[0, sl] * wk[0]
                acch = bufh[0, sl] * wk[_EMB_K]
                for k in range(1, _EMB_K):
                    accw = accw + bufw[k, sl] * wk[k]
                    acch = acch + bufh[k, sl] * wk[_EMB_K + k]
                sbw[sl] = accw
                sbh[sl] = acch
                return c2

            lax.fori_loop(0, _C // lanes, col_body, 0)
            pltpu.sync_copy(sbw, sw_hbm.at[pl.ds(u0, _C)])
            pltpu.sync_copy(sbh, sh_hbm.at[pl.ds(u0, _C)])
            return carry

        lax.fori_loop(0, n_chunks, chunk_body, 0)

    return fold


def _build_sc_gather():
    info = plsc.get_sparse_core_info()
    nc, ns, lanes = info.num_cores, info.num_subcores, info.num_lanes
    nw = nc * ns
    b_per_w = _BATCH // nw
    n_chunks = b_per_w // 128

    mesh = plsc.VectorSubcoreMesh(core_axis_name="c", subcore_axis_name="s")

    @functools.partial(
        pl.kernel,
        out_type=jax.ShapeDtypeStruct((_BATCH,), jnp.float32),
        mesh=mesh,
        compiler_params=pltpu.CompilerParams(
            needs_layout_passes=False, use_tc_tiling_on_sc=False),
        scratch_types=[
            pltpu.VMEM((n_chunks, 128), jnp.int32),
            pltpu.VMEM((n_chunks, 128), jnp.int32),
            pltpu.VMEM((n_chunks, 128), jnp.float32),
            pltpu.VMEM((n_chunks, 128), jnp.float32),
            pltpu.VMEM((n_chunks, 128), jnp.float32),
            pltpu.VMEM((n_chunks, 128), jnp.float32),
            pltpu.VMEM((16,), jnp.float32),
            pltpu.VMEM((b_per_w,), jnp.float32),
            pltpu.SemaphoreType.DMA,
        ],
    )
    def ncf(uid_hbm, vid_hbm, swt_hbm, sht_hbm, sws_hbm, shs_hbm, b_hbm,
            out_hbm, idx_u, idx_v, zut, zvt, zus, zvs, bvm, out_v, sem):
        wid = lax.axis_index("s") * nc + lax.axis_index("c")
        row0 = wid * n_chunks
        pltpu.sync_copy(uid_hbm.at[pl.ds(row0, n_chunks)], idx_u)
        pltpu.sync_copy(vid_hbm.at[pl.ds(row0, n_chunks)], idx_v)
        pltpu.sync_copy(b_hbm, bvm)

        copies = []
        for c in range(n_chunks):
            copies.append(pltpu.async_copy(swt_hbm.at[idx_u.at[c]],
                                           zut.at[c], sem))
            copies.append(pltpu.async_copy(sht_hbm.at[idx_v.at[c]],
                                           zvt.at[c], sem))
            copies.append(pltpu.async_copy(sws_hbm.at[idx_u.at[c]],
                                           zus.at[c], sem))
            copies.append(pltpu.async_copy(shs_hbm.at[idx_v.at[c]],
                                           zvs.at[c], sem))
        for cp in copies:
            cp.wait()

        bias = bvm[:]
        for c in range(n_chunks):
            for j in range(128 // lanes):
                sl = pl.ds(j * lanes, lanes)
                u = idx_u[c, sl]
                v = idx_v[c, sl]
                zu = jnp.where((u >= _A_BEG) & (u < _B_END),
                               zus[c, sl], zut[c, sl])
                zv = jnp.where((v >= _A_BEG) & (v < _B_END),
                               zvs[c, sl], zvt[c, sl])
                z = zu + zv + bias
                out_v[pl.ds((c * 128 + j * lanes), lanes)] = (
                    1.0 / (1.0 + jnp.exp(-z)))

        pltpu.sync_copy(out_v, out_hbm.at[pl.ds(wid * b_per_w, b_per_w)])

    return ncf


_SC_FOLD = _build_sc_fold()
_SC_GATHER = _build_sc_gather()


def kernel(x, W, H, lin_w, lin_b):
    uid = x[:, 0].astype(jnp.int32).reshape(128, 128)
    vid = x[:, 1].astype(jnp.int32).reshape(128, 128)
    lw = lin_w.reshape(2 * _EMB_K).astype(jnp.float32)
    wu = jnp.broadcast_to(lw[:_EMB_K, None], (_EMB_K, 128))
    wv = jnp.broadcast_to(lw[_EMB_K:, None], (_EMB_K, 128))
    wt4 = W.T.reshape(_KG, _KS, _ROWS)
    ht4 = H.T.reshape(_KG, _KS, _ROWS)
    swt, sht = _fold_tables_tc(W.T, H.T, wu, wv)
    sws, shs = _SC_FOLD(wt4, ht4, lw)
    bb = jnp.broadcast_to(lin_b.astype(jnp.float32).reshape(1), (16,))
    return _SC_GATHER(uid, vid, swt, sht, sws, shs, bb)


# speedup vs baseline: 1.1944x; 1.1944x over previous
import functools

import jax
import jax.numpy as jnp
from jax import lax
from jax.experimental import pallas as pl
from jax.experimental.pallas import tpu as pltpu
from jax.experimental.pallas import tpu_sc as plsc

_BATCH = 16384
_EMB_K = 32
_ROWS = 1000000
_BU = 32768
_NBT = (_ROWS + _BU - 1) // _BU
_SROWS = _NBT * _BU // 128
_SFLAT = _SROWS * 128

_C = 512
_KG = 4
_KS = 8
_B_END = 30 * _BU
_F_SC = 12 * 32 * _C
_A_BEG = _B_END - _F_SC


def _tc_fold(wt_ref, ht_ref, wu_ref, wv_ref, sw_ref, sh_ref):
    wu = wu_ref[...]
    wv = wv_ref[...]
    wb = wt_ref[...]
    hb = ht_ref[...]
    for j in range(_BU // 128):
        sl = slice(j * 128, (j + 1) * 128)
        sw_ref[j, :] = jnp.sum(wb[:, sl] * wu, axis=0)
        sh_ref[j, :] = jnp.sum(hb[:, sl] * wv, axis=0)


def _fold_tables_tc(wt, ht, wu, wv):
    n1 = _A_BEG // _BU
    n2 = _NBT - _B_END // _BU
    off = _B_END // _BU - n1

    def imap_in(i):
        j = jnp.where(i < n1, i, i + off)
        return (0, j)

    def imap_out(i):
        j = jnp.where(i < n1, i, i + off)
        return (j, 0)

    sw, sh = pl.pallas_call(
        _tc_fold,
        grid=(n1 + n2,),
        in_specs=[
            pl.BlockSpec((_EMB_K, _BU), imap_in),
            pl.BlockSpec((_EMB_K, _BU), imap_in),
            pl.BlockSpec((_EMB_K, 128), lambda i: (0, 0)),
            pl.BlockSpec((_EMB_K, 128), lambda i: (0, 0)),
        ],
        out_specs=[
            pl.BlockSpec((_BU // 128, 128), imap_out),
            pl.BlockSpec((_BU // 128, 128), imap_out),
        ],
        out_shape=[
            jax.ShapeDtypeStruct((_SROWS, 128), jnp.float32),
            jax.ShapeDtypeStruct((_SROWS, 128), jnp.float32),
        ],
        compiler_params=pltpu.CompilerParams(
            dimension_semantics=("arbitrary",)),
    )(wt, ht, wu, wv)
    return sw.reshape(_SFLAT), sh.reshape(_SFLAT)


def _build_sc_fold():
    info = plsc.get_sparse_core_info()
    nc, ns, lanes = info.num_cores, info.num_subcores, info.num_lanes
    nw = nc * ns
    per_w = _F_SC // nw
    n_chunks = per_w // _C

    mesh = plsc.VectorSubcoreMesh(core_axis_name="c", subcore_axis_name="s")

    @functools.partial(
        pl.kernel,
        out_type=[jax.ShapeDtypeStruct((_SFLAT,), jnp.float32),
                  jax.ShapeDtypeStruct((_SFLAT,), jnp.float32)],
        mesh=mesh,
        compiler_params=pltpu.CompilerParams(
            needs_layout_passes=False, use_tc_tiling_on_sc=True),
        scratch_types=[
            pltpu.VMEM((2, _KG * _KS, _C), jnp.float32),
            pltpu.VMEM((2, _KG * _KS, _C), jnp.float32),
            pltpu.VMEM((2 * _EMB_K,), jnp.float32),
            pltpu.VMEM((_C,), jnp.float32),
            pltpu.VMEM((_C,), jnp.float32),
            pltpu.SemaphoreType.DMA,
            pltpu.SemaphoreType.DMA,
        ],
    )
    def fold(wt4_hbm, ht4_hbm, w_hbm, sw_hbm, sh_hbm,
             bufw, bufh, wvm, sbw, sbh, sem0, sem1):
        wid = lax.axis_index("s") * nc + lax.axis_index("c")
        base = _A_BEG + wid * per_w
        pltpu.sync_copy(w_hbm, wvm)
        wregs = [wvm[pl.ds(j * lanes, lanes)]
                 for j in range(2 * _EMB_K // lanes)]
        wk = [wregs[k // lanes][k % lanes] for k in range(2 * _EMB_K)]
        sems = [sem0, sem1]

        def fire(ci, b):
            u0 = base + ci * _C
            for g in range(_KG):
                pltpu.async_copy(wt4_hbm.at[g, :, pl.ds(u0, _C)],
                                 bufw.at[b, pl.ds(g * _KS, _KS)], sems[b])
                pltpu.async_copy(ht4_hbm.at[g, :, pl.ds(u0, _C)],
                                 bufh.at[b, pl.ds(g * _KS, _KS)], sems[b])

        def drain(b):
            for g in range(_KG):
                pltpu.make_async_copy(
                    wt4_hbm.at[0, :, pl.ds(base, _C)],
                    bufw.at[b, pl.ds(g * _KS, _KS)], sems[b]).wait()
                pltpu.make_async_copy(
                    ht4_hbm.at[0, :, pl.ds(base, _C)],
                    bufh.at[b, pl.ds(g * _KS, _KS)], sems[b]).wait()

        def compute(ci, b):
            u0 = base + ci * _C

            def col_body(j, c2):
                sl = pl.ds(j * lanes, lanes)
                accw = bufw[b, 0, sl] * wk[0]
                acch = bufh[b, 0, sl] * wk[_EMB_K]
                for k in range(1, _EMB_K):
                    accw = accw + bufw[b, k, sl] * wk[k]
                    acch = acch + bufh[b, k, sl] * wk[_EMB_K + k]
                sbw[sl] = accw
                sbh[sl] = acch
                return c2

            lax.fori_loop(0, _C // lanes, col_body, 0)
            pltpu.sync_copy(sbw, sw_hbm.at[pl.ds(u0, _C)])
            pltpu.sync_copy(sbh, sh_hbm.at[pl.ds(u0, _C)])

        fire(0, 0)

        def pair_body(i2, carry):
            cp = i2 * 2
            fire(cp + 1, 1)
            drain(0)
            compute(cp, 0)

            @pl.when(cp + 2 < n_chunks)
            def _():
                fire(cp + 2, 0)

            drain(1)
            compute(cp + 1, 1)
            return carry

        lax.fori_loop(0, n_chunks // 2, pair_body, 0)

    return fold


def _build_sc_gather():
    info = plsc.get_sparse_core_info()
    nc, ns, lanes = info.num_cores, info.num_subcores, info.num_lanes
    nw = nc * ns
    b_per_w = _BATCH // nw
    n_chunks = b_per_w // 128

    mesh = plsc.VectorSubcoreMesh(core_axis_name="c", subcore_axis_name="s")

    @functools.partial(
        pl.kernel,
        out_type=jax.ShapeDtypeStruct((_BATCH,), jnp.float32),
        mesh=mesh,
        compiler_params=pltpu.CompilerParams(
            needs_layout_passes=False, use_tc_tiling_on_sc=False),
        scratch_types=[
            pltpu.VMEM((n_chunks, 128), jnp.int32),
            pltpu.VMEM((n_chunks, 128), jnp.int32),
            pltpu.VMEM((n_chunks, 128), jnp.float32),
            pltpu.VMEM((n_chunks, 128), jnp.float32),
            pltpu.VMEM((n_chunks, 128), jnp.float32),
            pltpu.VMEM((n_chunks, 128), jnp.float32),
            pltpu.VMEM((16,), jnp.float32),
            pltpu.VMEM((b_per_w,), jnp.float32),
            pltpu.SemaphoreType.DMA,
        ],
    )
    def ncf(uid_hbm, vid_hbm, swt_hbm, sht_hbm, sws_hbm, shs_hbm, b_hbm,
            out_hbm, idx_u, idx_v, zut, zvt, zus, zvs, bvm, out_v, sem):
        wid = lax.axis_index("s") * nc + lax.axis_index("c")
        row0 = wid * n_chunks
        pltpu.sync_copy(uid_hbm.at[pl.ds(row0, n_chunks)], idx_u)
        pltpu.sync_copy(vid_hbm.at[pl.ds(row0, n_chunks)], idx_v)
        pltpu.sync_copy(b_hbm, bvm)

        copies = []
        for c in range(n_chunks):
            copies.append(pltpu.async_copy(swt_hbm.at[idx_u.at[c]],
                                           zut.at[c], sem))
            copies.append(pltpu.async_copy(sht_hbm.at[idx_v.at[c]],
                                           zvt.at[c], sem))
            copies.append(pltpu.async_copy(sws_hbm.at[idx_u.at[c]],
                                           zus.at[c], sem))
            copies.append(pltpu.async_copy(shs_hbm.at[idx_v.at[c]],
                                           zvs.at[c], sem))
        for cp in copies:
            cp.wait()

        bias = bvm[:]
        for c in range(n_chunks):
            for j in range(128 // lanes):
                sl = pl.ds(j * lanes, lanes)
                u = idx_u[c, sl]
                v = idx_v[c, sl]
                zu = jnp.where((u >= _A_BEG) & (u < _B_END),
                               zus[c, sl], zut[c, sl])
                zv = jnp.where((v >= _A_BEG) & (v < _B_END),
                               zvs[c, sl], zvt[c, sl])
                z = zu + zv + bias
                out_v[pl.ds((c * 128 + j * lanes), lanes)] = (
                    1.0 / (1.0 + jnp.exp(-z)))

        pltpu.sync_copy(out_v, out_hbm.at[pl.ds(wid * b_per_w, b_per_w)])

    return ncf


_SC_FOLD = _build_sc_fold()
_SC_GATHER = _build_sc_gather()


def kernel(x, W, H, lin_w, lin_b):
    uid = x[:, 0].astype(jnp.int32).reshape(128, 128)
    vid = x[:, 1].astype(jnp.int32).reshape(128, 128)
    lw = lin_w.reshape(2 * _EMB_K).astype(jnp.float32)
    wu = jnp.broadcast_to(lw[:_EMB_K, None], (_EMB_K, 128))
    wv = jnp.broadcast_to(lw[_EMB_K:, None], (_EMB_K, 128))
    wt4 = W.T.reshape(_KG, _KS, _ROWS)
    ht4 = H.T.reshape(_KG, _KS, _ROWS)
    swt, sht = _fold_tables_tc(W.T, H.T, wu, wv)
    sws, shs = _SC_FOLD(wt4, ht4, lw)
    bb = jnp.broadcast_to(lin_b.astype(jnp.float32).reshape(1), (16,))
    return _SC_GATHER(uid, vid, swt, sht, sws, shs, bb)


# speedup vs baseline: 1.2213x; 1.0226x over previous
import functools

import jax
import jax.numpy as jnp
from jax import lax
from jax.experimental import pallas as pl
from jax.experimental.pallas import tpu as pltpu
from jax.experimental.pallas import tpu_sc as plsc

_BATCH = 16384
_EMB_K = 32
_ROWS = 1000000
_BU = 32768
_NBLK = (_ROWS + _BU - 1) // _BU
_SROWS = _NBLK * _BU // 128


def _tc_fold(wt_ref, ht_ref, wu_ref, wv_ref, sw_ref, sh_ref):
    wu = wu_ref[...]
    wv = wv_ref[...]
    wb = wt_ref[...]
    hb = ht_ref[...]
    for j in range(_BU // 128):
        sl = slice(j * 128, (j + 1) * 128)
        sw_ref[j, :] = jnp.sum(wb[:, sl] * wu, axis=0)
        sh_ref[j, :] = jnp.sum(hb[:, sl] * wv, axis=0)


def _fold_tables(wt, ht, wu, wv):
    grid = (_NBLK,)
    out_shape = [
        jax.ShapeDtypeStruct((_SROWS, 128), jnp.float32),
        jax.ShapeDtypeStruct((_SROWS, 128), jnp.float32),
    ]
    sw, sh = pl.pallas_call(
        _tc_fold,
        grid=grid,
        in_specs=[
            pl.BlockSpec((_EMB_K, _BU), lambda i: (0, i)),
            pl.BlockSpec((_EMB_K, _BU), lambda i: (0, i)),
            pl.BlockSpec((_EMB_K, 128), lambda i: (0, 0)),
            pl.BlockSpec((_EMB_K, 128), lambda i: (0, 0)),
        ],
        out_specs=[
            pl.BlockSpec((_BU // 128, 128), lambda i: (i, 0)),
            pl.BlockSpec((_BU // 128, 128), lambda i: (i, 0)),
        ],
        out_shape=out_shape,
        compiler_params=pltpu.CompilerParams(
            dimension_semantics=("arbitrary",)),
    )(wt, ht, wu, wv)
    return sw.reshape(_SROWS * 128), sh.reshape(_SROWS * 128)


def _build_sc():
    info = plsc.get_sparse_core_info()
    nc, ns, lanes = info.num_cores, info.num_subcores, info.num_lanes
    nw = nc * ns
    b_per_w = _BATCH // nw
    n_chunks = b_per_w // 128

    mesh = plsc.VectorSubcoreMesh(core_axis_name="c", subcore_axis_name="s")

    @functools.partial(
        pl.kernel,
        out_type=jax.ShapeDtypeStruct((_BATCH,), jnp.float32),
        mesh=mesh,
        compiler_params=pltpu.CompilerParams(
            needs_layout_passes=False, use_tc_tiling_on_sc=False),
        scratch_types=[
            pltpu.VMEM((n_chunks, 128), jnp.int32),
            pltpu.VMEM((n_chunks, 128), jnp.int32),
            pltpu.VMEM((n_chunks, 128), jnp.float32),
            pltpu.VMEM((n_chunks, 128), jnp.float32),
            pltpu.VMEM((16,), jnp.float32),
            pltpu.VMEM((b_per_w,), jnp.float32),
            pltpu.SemaphoreType.DMA,
        ],
    )
    def ncf(uid_hbm, vid_hbm, sw_hbm, sh_hbm, b_hbm, out_hbm,
            idx_u, idx_v, zu, zv, bvm, out_v, sem):
        wid = lax.axis_index("s") * nc + lax.axis_index("c")
        row0 = wid * n_chunks
        pltpu.sync_copy(uid_hbm.at[pl.ds(row0, n_chunks)], idx_u)
        pltpu.sync_copy(vid_hbm.at[pl.ds(row0, n_chunks)], idx_v)
        pltpu.sync_copy(b_hbm, bvm)

        copies = []
        for c in range(n_chunks):
            copies.append(pltpu.async_copy(sw_hbm.at[idx_u.at[c]],
                                           zu.at[c], sem))
            copies.append(pltpu.async_copy(sh_hbm.at[idx_v.at[c]],
                                           zv.at[c], sem))
        for cp in copies:
            cp.wait()

        bias = bvm[:]
        for c in range(n_chunks):
            for j in range(128 // lanes):
                z = (zu[c, pl.ds(j * lanes, lanes)]
                     + zv[c, pl.ds(j * lanes, lanes)] + bias)
                out_v[pl.ds((c * 128 + j * lanes), lanes)] = (
                    1.0 / (1.0 + jnp.exp(-z)))

        pltpu.sync_copy(out_v, out_hbm.at[pl.ds(wid * b_per_w, b_per_w)])

    return ncf


_NCF_SC = _build_sc()


def kernel(x, W, H, lin_w, lin_b):
    uid = x[:, 0].astype(jnp.int32).reshape(128, 128)
    vid = x[:, 1].astype(jnp.int32).reshape(128, 128)
    lw = lin_w.reshape(2 * _EMB_K).astype(jnp.float32)
    wu = jnp.broadcast_to(lw[:_EMB_K, None], (_EMB_K, 128))
    wv = jnp.broadcast_to(lw[_EMB_K:, None], (_EMB_K, 128))
    sw, sh = _fold_tables(W.T, H.T, wu, wv)
    bb = jnp.broadcast_to(lin_b.astype(jnp.float32).reshape(1), (16,))
    return _NCF_SC(uid, vid, sw, sh, bb)
